# baseline (speedup 1.0000x reference)
import functools

import jax
import jax.numpy as jnp
from jax import lax
from jax.experimental import pallas as pl
from jax.experimental.pallas import tpu as pltpu
from jax.experimental.pallas import tpu_sc as plsc

ROWS, COLS = 32, 8192
D = 64
NC, NS = 2, 16
NW = NC * NS
B_PER_W = ROWS * COLS // NW
CHUNK = 128
N_CHUNKS = B_PER_W // CHUNK
DT = 8

_mesh = plsc.VectorSubcoreMesh(core_axis_name="c", subcore_axis_name="s")


@functools.partial(
    pl.kernel,
    mesh=_mesh,
    out_type=jax.ShapeDtypeStruct((NW, DT, N_CHUNKS, 8, 128), jnp.float32),
    scratch_types=[
        pltpu.VMEM((B_PER_W,), jnp.int32),
        pltpu.VMEM((CHUNK, D), jnp.float32),
        pltpu.VMEM((DT, 8, 128), jnp.float32),
        pltpu.SemaphoreType.DMA,
    ],
    compiler_params=pltpu.CompilerParams(
        use_tc_tiling_on_sc=False, needs_layout_passes=False),
)
def _gather_kernel(x_hbm, table_hbm, out_hbm, idx_v, rows_v, tbuf, sem):
    wid = lax.axis_index("s") * NC + lax.axis_index("c")
    pltpu.sync_copy(x_hbm.at[wid], idx_v)

    def chunk_body(j, carry):
        pltpu.async_copy(
            table_hbm.at[idx_v.at[pl.ds(j * CHUNK, CHUNK)]], rows_v,
            sem).wait()
        def d_body(dd, c2):
            dt = dd // 8
            di = dd % 8
            for k in range(8):
                vec = plsc.load_gather(
                    rows_v, [lax.iota(jnp.int32, 16) + 16 * k,
                             jnp.full((16,), dd, jnp.int32)])
                tbuf[dt, di, pl.ds(16 * k, 16)] = vec
            return c2
        lax.fori_loop(0, D, d_body, 0)
        for dt in range(DT):
            pltpu.sync_copy(tbuf.at[dt], out_hbm.at[wid, dt, j])
        return carry

    lax.fori_loop(0, N_CHUNKS, chunk_body, 0)


def kernel(x, table):
    out5 = _gather_kernel(x, table)
    return out5.transpose(0, 2, 4, 1, 3).reshape(ROWS, COLS, D)
